# Initial kernel scaffold; baseline (speedup 1.0000x reference)
import jax
import jax.numpy as jnp
from jax.experimental import pallas as pl


def kernel(x, emb_table, W, b):
    raise NotImplementedError("write your pallas kernel here")


# speedup vs baseline: 1.0466x; 1.0466x over previous
import functools

import jax
import jax.numpy as jnp
from jax import lax
from jax.experimental import pallas as pl
from jax.experimental.pallas import tpu as pltpu
from jax.experimental.pallas import tpu_sc as plsc

_B = 16384
_V = 26
_NC = 2
_NS = 16
_NW = _NC * _NS
_BPW = _B // _NW
_L = 16
_NG = _BPW // _L


def _table_body(emb_ref, w_ref, b_ref, out_ref):
    logits = lax.dot_general(
        emb_ref[...], w_ref[...], (((1,), (1,)), ((), ())),
        preferred_element_type=jnp.float32,
    ) + b_ref[...]
    m = jnp.max(logits, axis=1, keepdims=True)
    s = jnp.sum(jnp.exp(logits - m), axis=1, keepdims=True)
    out_ref[...] = logits - m - jnp.log(s)


_table_call = pl.pallas_call(
    _table_body,
    out_shape=jax.ShapeDtypeStruct((_V, _V), jnp.float32),
)


@functools.partial(
    pl.kernel,
    out_type=jax.ShapeDtypeStruct((_B * _V,), jnp.float32),
    mesh=plsc.VectorSubcoreMesh(core_axis_name="c", subcore_axis_name="s"),
    compiler_params=pltpu.CompilerParams(needs_layout_passes=False),
    scratch_types=[
        pltpu.VMEM((_V * _V,), jnp.float32),
        pltpu.VMEM((_BPW,), jnp.int32),
        pltpu.VMEM((_BPW * _V,), jnp.float32),
    ],
)
def _gather_call(table_hbm, x_hbm, out_hbm, table_v, idx_v, rows_v):
    wid = lax.axis_index("s") * _NC + lax.axis_index("c")
    base = wid * _BPW
    pltpu.sync_copy(table_hbm, table_v)
    pltpu.sync_copy(x_hbm.at[pl.ds(base, _BPW)], idx_v)
    lanes26 = lax.iota(jnp.int32, _L) * _V

    def grp(g, carry):
        idx16 = idx_v[pl.ds(g * _L, _L)]
        tpos = idx16 * _V
        opos = g * (_L * _V) + lanes26
        for j in range(_V):
            vals = plsc.load_gather(table_v, [tpos + j])
            plsc.store_scatter(rows_v, [opos + j], vals)
        return carry

    lax.fori_loop(0, _NG, grp, 0)
    pltpu.sync_copy(rows_v, out_hbm.at[pl.ds(base * _V, _BPW * _V)])


def kernel(x, emb_table, W, b):
    table = _table_call(emb_table, W, b.reshape(1, _V))
    out_flat = _gather_call(table.reshape(_V * _V), x.astype(jnp.int32))
    return out_flat.reshape(_B, _V)


# speedup vs baseline: 1.0982x; 1.0492x over previous
import functools

import jax
import jax.numpy as jnp
from jax import lax
from jax.experimental import pallas as pl
from jax.experimental.pallas import tpu as pltpu
from jax.experimental.pallas import tpu_sc as plsc

_B = 16384
_V = 26
_H = 5
_NC = 2
_NS = 16
_NW = _NC * _NS
_BPW = _B // _NW
_L = 16
_NG = _BPW // _L
_VP = 32

_LN2_SCALE = float(jnp.log(2.0)) / (1 << 23)
_ONE_BITS = float(0x3F800000)


@functools.partial(
    pl.kernel,
    out_type=jax.ShapeDtypeStruct((_B * _V,), jnp.float32),
    mesh=plsc.VectorSubcoreMesh(core_axis_name="c", subcore_axis_name="s"),
    compiler_params=pltpu.CompilerParams(needs_layout_passes=False),
    scratch_types=[
        pltpu.VMEM((_V * _H,), jnp.float32),
        pltpu.VMEM((_V * _H + 8,), jnp.float32),
        pltpu.VMEM((_V + 8,), jnp.float32),
        pltpu.VMEM((_V * _VP,), jnp.float32),
        pltpu.VMEM((_BPW,), jnp.int32),
        pltpu.VMEM((_BPW * _V,), jnp.float32),
        pltpu.SemaphoreType.DMA,
    ],
)
def _fused_call(emb_hbm, w_hbm, b_hbm, x_hbm, out_hbm,
                emb_v, w_v, b_v, tab_v, idx_v, rows_v, sem):
    wid = lax.axis_index("s") * _NC + lax.axis_index("c")
    base = wid * _BPW
    copies = [
        pltpu.async_copy(emb_hbm, emb_v, sem),
        pltpu.async_copy(w_hbm, w_v.at[pl.ds(8, _V * _H)], sem),
        pltpu.async_copy(b_hbm, b_v.at[pl.ds(8, _V)], sem),
        pltpu.async_copy(x_hbm.at[pl.ds(base, _BPW)], idx_v, sem),
    ]
    for c in copies:
        c.wait()

    lanes = lax.iota(jnp.int32, _L)

    embk = []
    for v in range(2):
        rows = jnp.minimum(v * _L + lanes, _V - 1) * _H
        embk.append([plsc.load_gather(emb_v, [rows + k]) for k in range(_H)])

    m = [None, None]
    for j in range(_V):
        bj = plsc.load_gather(b_v, [jnp.full((_L,), 8 + j, jnp.int32)])
        ws = [
            plsc.load_gather(
                w_v, [jnp.full((_L,), 8 + j * _H + k, jnp.int32)])
            for k in range(_H)
        ]
        for v in range(2):
            col = embk[v][0] * ws[0]
            for k in range(1, _H):
                col = col + embk[v][k] * ws[k]
            col = col + bj
            tab_v[pl.ds(j * _VP + v * _L, _L)] = col
            m[v] = col if m[v] is None else jnp.maximum(m[v], col)

    s = [jnp.zeros((_L,), jnp.float32), jnp.zeros((_L,), jnp.float32)]
    for j in range(_V):
        for v in range(2):
            col = tab_v[pl.ds(j * _VP + v * _L, _L)]
            s[v] = s[v] + jnp.exp(col - m[v])

    lse = []
    for v in range(2):
        sv = s[v]
        y = (plsc.bitcast(sv, jnp.int32).astype(jnp.float32) - _ONE_BITS) \
            * _LN2_SCALE
        for _ in range(3):
            y = y - 1.0 + sv * jnp.exp(-y)
        lse.append(y + m[v])

    for j in range(_V):
        for v in range(2):
            sl = pl.ds(j * _VP + v * _L, _L)
            tab_v[sl] = tab_v[sl] - lse[v]

    lanes26 = lanes * _V

    def grp(g, carry):
        idx16 = idx_v[pl.ds(g * _L, _L)]
        opos = g * (_L * _V) + lanes26
        for j in range(_V):
            vals = plsc.load_gather(tab_v, [idx16 + j * _VP])
            plsc.store_scatter(rows_v, [opos + j], vals)
        return carry

    lax.fori_loop(0, _NG, grp, 0)
    pltpu.sync_copy(rows_v, out_hbm.at[pl.ds(base * _V, _BPW * _V)])


def kernel(x, emb_table, W, b):
    out_flat = _fused_call(
        emb_table.reshape(_V * _H),
        W.reshape(_V * _H),
        b,
        x.astype(jnp.int32),
    )
    return out_flat.reshape(_B, _V)


# speedup vs baseline: 1.2510x; 1.1392x over previous
import functools

import jax
import jax.numpy as jnp
from jax import lax
from jax.experimental import pallas as pl
from jax.experimental.pallas import tpu as pltpu
from jax.experimental.pallas import tpu_sc as plsc

_B = 16384
_V = 26
_H = 5
_NC = 2
_NS = 16
_NW = _NC * _NS
_BPW = _B // _NW
_L = 16
_NG = _BPW // _L
_VP = 32

_LN2_SCALE = float(jnp.log(2.0)) / (1 << 23)
_ONE_BITS = float(0x3F800000)


@functools.partial(
    pl.kernel,
    out_type=jax.ShapeDtypeStruct((_B, _V), jnp.float32),
    mesh=plsc.VectorSubcoreMesh(core_axis_name="c", subcore_axis_name="s"),
    compiler_params=pltpu.CompilerParams(needs_layout_passes=False),
    scratch_types=[
        pltpu.VMEM((_V * _H,), jnp.float32),
        pltpu.VMEM((_V * _H + 8,), jnp.float32),
        pltpu.VMEM((_V + 8,), jnp.float32),
        pltpu.VMEM((_V * _VP,), jnp.float32),
        pltpu.VMEM((_BPW,), jnp.int32),
        pltpu.VMEM((_BPW, _V), jnp.float32),
        pltpu.SemaphoreType.DMA,
    ],
)
def _fused_call(emb_hbm, w_hbm, b_hbm, x_hbm, out_hbm,
                emb_v, w_v, b_v, tab_v, idx_v, rows_v, sem):
    wid = lax.axis_index("s") * _NC + lax.axis_index("c")
    base = wid * _BPW
    copies = [
        pltpu.async_copy(emb_hbm, emb_v, sem),
        pltpu.async_copy(w_hbm, w_v.at[pl.ds(8, _V * _H)], sem),
        pltpu.async_copy(b_hbm, b_v.at[pl.ds(8, _V)], sem),
        pltpu.async_copy(x_hbm.at[pl.ds(base, _BPW)], idx_v, sem),
    ]
    for c in copies:
        c.wait()

    lanes = lax.iota(jnp.int32, _L)

    embk = []
    for v in range(2):
        rows = jnp.minimum(v * _L + lanes, _V - 1) * _H
        embk.append([plsc.load_gather(emb_v, [rows + k]) for k in range(_H)])

    m = [None, None]
    for j in range(_V):
        bj = plsc.load_gather(b_v, [jnp.full((_L,), 8 + j, jnp.int32)])
        ws = [
            plsc.load_gather(
                w_v, [jnp.full((_L,), 8 + j * _H + k, jnp.int32)])
            for k in range(_H)
        ]
        for v in range(2):
            col = embk[v][0] * ws[0]
            for k in range(1, _H):
                col = col + embk[v][k] * ws[k]
            col = col + bj
            tab_v[pl.ds(j * _VP + v * _L, _L)] = col
            m[v] = col if m[v] is None else jnp.maximum(m[v], col)

    s = [jnp.zeros((_L,), jnp.float32), jnp.zeros((_L,), jnp.float32)]
    for j in range(_V):
        for v in range(2):
            col = tab_v[pl.ds(j * _VP + v * _L, _L)]
            s[v] = s[v] + jnp.exp(col - m[v])

    lse = []
    for v in range(2):
        sv = s[v]
        y = (plsc.bitcast(sv, jnp.int32).astype(jnp.float32) - _ONE_BITS) \
            * _LN2_SCALE
        for _ in range(3):
            y = y - 1.0 + sv * jnp.exp(-y)
        lse.append(y + m[v])

    for j in range(_V):
        for v in range(2):
            sl = pl.ds(j * _VP + v * _L, _L)
            tab_v[sl] = tab_v[sl] - lse[v]

    def grp(g, carry):
        idx16 = idx_v[pl.ds(g * _L, _L)]
        rows16 = g * _L + lanes
        for j in range(_V):
            vals = plsc.load_gather(tab_v, [idx16 + j * _VP])
            plsc.store_scatter(
                rows_v, [rows16, jnp.full((_L,), j, jnp.int32)], vals)
        return carry

    lax.fori_loop(0, _NG, grp, 0)
    pltpu.sync_copy(rows_v, out_hbm.at[pl.ds(base, _BPW)])


def kernel(x, emb_table, W, b):
    return _fused_call(
        emb_table.reshape(_V * _H),
        W.reshape(_V * _H),
        b,
        x.astype(jnp.int32),
    )
